# initial kernel scaffold (unmeasured)
import jax
import jax.numpy as jnp
from jax import lax
from jax.experimental import pallas as pl
from jax.experimental.pallas import tpu as pltpu

M_HALF = 512
F = 4096


def kernel(x, dy):
    def body(x_ref, dy_ref, out_ref, send_ref, recv_ref, send_sem, recv_sem):
        my_x = lax.axis_index("x")
        my_y = lax.axis_index("y")
        my_z = lax.axis_index("z")

        barrier_sem = pltpu.get_barrier_semaphore()
        pl.semaphore_signal(
            barrier_sem, inc=1,
            device_id=(my_x, 1 - my_y, my_z),
            device_id_type=pl.DeviceIdType.MESH,
        )
        pl.semaphore_wait(barrier_sem, 1)

        x_bf = x_ref[...].astype(jnp.bfloat16)
        dy_bf = dy_ref[...].astype(jnp.bfloat16)

        dot = lambda a, b: lax.dot_general(
            a, b, (((0,), (0,)), ((), ())),
            preferred_element_type=jnp.float32,
        )

        @pl.when(my_y == 0)
        def _():
            send_ref[...] = dot(x_bf[:, M_HALF:], dy_bf).astype(jnp.bfloat16)

        @pl.when(my_y == 1)
        def _():
            send_ref[...] = dot(x_bf[:, :M_HALF], dy_bf).astype(jnp.bfloat16)

        rdma = pltpu.make_async_remote_copy(
            src_ref=send_ref,
            dst_ref=recv_ref,
            send_sem=send_sem,
            recv_sem=recv_sem,
            device_id=(my_x, 1 - my_y, my_z),
            device_id_type=pl.DeviceIdType.MESH,
        )
        rdma.start()

        @pl.when(my_y == 0)
        def _():
            out_ref[...] = dot(x_bf[:, :M_HALF], dy_bf)

        @pl.when(my_y == 1)
        def _():
            out_ref[...] = dot(x_bf[:, M_HALF:], dy_bf)

        rdma.wait()
        out_ref[...] = out_ref[...] + recv_ref[...].astype(jnp.float32)

    m_half = x.shape[1] // 2
    return pl.pallas_call(
        body,
        out_shape=jax.ShapeDtypeStruct((m_half, dy.shape[1]), jnp.float32),
        in_specs=[
            pl.BlockSpec(memory_space=pltpu.VMEM),
            pl.BlockSpec(memory_space=pltpu.VMEM),
        ],
        out_specs=pl.BlockSpec(memory_space=pltpu.VMEM),
        scratch_shapes=[
            pltpu.VMEM((m_half, dy.shape[1]), jnp.bfloat16),
            pltpu.VMEM((m_half, dy.shape[1]), jnp.bfloat16),
            pltpu.SemaphoreType.DMA,
            pltpu.SemaphoreType.DMA,
        ],
        compiler_params=pltpu.CompilerParams(collective_id=0),
    )(x, dy)


# baseline (device time: 71333 ns/iter reference)
import jax
import jax.numpy as jnp
from jax import lax
from jax.experimental import pallas as pl
from jax.experimental.pallas import tpu as pltpu

M_HALF = 512
F = 4096


def kernel(x, dy):
    def body(x_ref, dy_ref, out_ref, send_ref, recv_ref, send_sem, recv_sem):
        my_x = lax.axis_index("x")
        my_y = lax.axis_index("y")
        my_z = lax.axis_index("z")

        barrier_sem = pltpu.get_barrier_semaphore()
        pl.semaphore_signal(
            barrier_sem, inc=1,
            device_id=(my_x, 1 - my_y, my_z),
            device_id_type=pl.DeviceIdType.MESH,
        )
        pl.semaphore_wait(barrier_sem, 1)

        x_bf = x_ref[...].astype(jnp.bfloat16)
        dy_bf = dy_ref[...].astype(jnp.bfloat16)

        dot = lambda a, b: lax.dot_general(
            a, b, (((0,), (0,)), ((), ())),
            preferred_element_type=jnp.float32,
        )

        @pl.when(my_y == 0)
        def _():
            send_ref[...] = dot(x_bf[:, M_HALF:], dy_bf).astype(jnp.bfloat16)

        @pl.when(my_y == 1)
        def _():
            send_ref[...] = dot(x_bf[:, :M_HALF], dy_bf).astype(jnp.bfloat16)

        rdma = pltpu.make_async_remote_copy(
            src_ref=send_ref,
            dst_ref=recv_ref,
            send_sem=send_sem,
            recv_sem=recv_sem,
            device_id=(my_x, 1 - my_y, my_z),
            device_id_type=pl.DeviceIdType.MESH,
        )
        rdma.start()

        @pl.when(my_y == 0)
        def _():
            out_ref[...] = dot(x_bf[:, :M_HALF], dy_bf)

        @pl.when(my_y == 1)
        def _():
            out_ref[...] = dot(x_bf[:, M_HALF:], dy_bf)

        rdma.wait()
        out_ref[...] = out_ref[...] + recv_ref[...].astype(jnp.float32)

    m_half = x.shape[1] // 2
    return pl.pallas_call(
        body,
        out_shape=jax.ShapeDtypeStruct((m_half, dy.shape[1]), jnp.float32),
        in_specs=[
            pl.BlockSpec(memory_space=pltpu.VMEM),
            pl.BlockSpec(memory_space=pltpu.VMEM),
        ],
        out_specs=pl.BlockSpec(memory_space=pltpu.VMEM),
        scratch_shapes=[
            pltpu.VMEM((m_half, dy.shape[1]), jnp.bfloat16),
            pltpu.VMEM((m_half, dy.shape[1]), jnp.bfloat16),
            pltpu.SemaphoreType.DMA,
            pltpu.SemaphoreType.DMA,
        ],
        compiler_params=pltpu.CompilerParams(
            collective_id=0, vmem_limit_bytes=100 * 1024 * 1024
        ),
    )(x, dy)


# device time: 66413 ns/iter; 1.0741x vs baseline; 1.0741x over previous
import jax
import jax.numpy as jnp
from jax import lax
from jax.experimental import pallas as pl
from jax.experimental.pallas import tpu as pltpu

N_CHUNK = 8
M_HALF = 512
FC = 512


def kernel(x, dy):
    def body(
        x_ref, dy_ref, out_ref,
        dyc_ref, xk_ref, xs_ref,
        p1_send, p1_recv, p1_send_sem, p1_recv_sem,
        gather_buf, g_send_sems, g_recv_sems,
    ):
        my_x = lax.axis_index("x")
        my_y = lax.axis_index("y")
        my_z = lax.axis_index("z")
        my_c = my_z * 2 + my_x

        barrier_sem = pltpu.get_barrier_semaphore()
        pl.semaphore_signal(
            barrier_sem, inc=1,
            device_id=(my_x, 1 - my_y, my_z),
            device_id_type=pl.DeviceIdType.MESH,
        )
        for k in range(1, N_CHUNK):
            pc = (my_c + k) % N_CHUNK
            pl.semaphore_signal(
                barrier_sem, inc=1,
                device_id=(pc % 2, my_y, pc // 2),
                device_id_type=pl.DeviceIdType.MESH,
            )
        pl.semaphore_wait(barrier_sem, N_CHUNK)

        dyc_ref[...] = dy_ref[:, pl.ds(my_c * FC, FC)].astype(jnp.bfloat16)

        @pl.when(my_y == 0)
        def _():
            xk_ref[...] = x_ref[:, :M_HALF].astype(jnp.bfloat16)
            xs_ref[...] = x_ref[:, M_HALF:].astype(jnp.bfloat16)

        @pl.when(my_y == 1)
        def _():
            xk_ref[...] = x_ref[:, M_HALF:].astype(jnp.bfloat16)
            xs_ref[...] = x_ref[:, :M_HALF].astype(jnp.bfloat16)

        dot = lambda a, b: lax.dot_general(
            a, b, (((0,), (0,)), ((), ())),
            preferred_element_type=jnp.float32,
        )

        p1_send[...] = dot(xs_ref[...], dyc_ref[...]).astype(jnp.bfloat16)
        rdma1 = pltpu.make_async_remote_copy(
            src_ref=p1_send, dst_ref=p1_recv,
            send_sem=p1_send_sem, recv_sem=p1_recv_sem,
            device_id=(my_x, 1 - my_y, my_z),
            device_id_type=pl.DeviceIdType.MESH,
        )
        rdma1.start()
        own = dot(xk_ref[...], dyc_ref[...])
        rdma1.wait()
        gather_buf[my_c, :, :] = (own + p1_recv[...].astype(jnp.float32)).astype(
            jnp.bfloat16
        )

        sends = []
        for k in range(1, N_CHUNK):
            pc = (my_c + k) % N_CHUNK
            rdma = pltpu.make_async_remote_copy(
                src_ref=gather_buf.at[my_c],
                dst_ref=gather_buf.at[my_c],
                send_sem=g_send_sems.at[pc],
                recv_sem=g_recv_sems.at[my_c],
                device_id=(pc % 2, my_y, pc // 2),
                device_id_type=pl.DeviceIdType.MESH,
            )
            rdma.start()
            sends.append(rdma)

        for k in range(1, N_CHUNK):
            pc = (my_c + k) % N_CHUNK
            recv = pltpu.make_async_remote_copy(
                src_ref=gather_buf.at[pc],
                dst_ref=gather_buf.at[pc],
                send_sem=g_send_sems.at[pc],
                recv_sem=g_recv_sems.at[pc],
                device_id=(pc % 2, my_y, pc // 2),
                device_id_type=pl.DeviceIdType.MESH,
            )
            recv.wait_recv()
        for s in sends:
            s.wait_send()

        for c in range(N_CHUNK):
            out_ref[:, c * FC:(c + 1) * FC] = gather_buf[c].astype(jnp.float32)

    k_dim = x.shape[0]
    return pl.pallas_call(
        body,
        out_shape=jax.ShapeDtypeStruct((M_HALF, dy.shape[1]), jnp.float32),
        in_specs=[
            pl.BlockSpec(memory_space=pltpu.VMEM),
            pl.BlockSpec(memory_space=pltpu.VMEM),
        ],
        out_specs=pl.BlockSpec(memory_space=pltpu.VMEM),
        scratch_shapes=[
            pltpu.VMEM((k_dim, FC), jnp.bfloat16),
            pltpu.VMEM((k_dim, M_HALF), jnp.bfloat16),
            pltpu.VMEM((k_dim, M_HALF), jnp.bfloat16),
            pltpu.VMEM((M_HALF, FC), jnp.bfloat16),
            pltpu.VMEM((M_HALF, FC), jnp.bfloat16),
            pltpu.SemaphoreType.DMA,
            pltpu.SemaphoreType.DMA,
            pltpu.VMEM((N_CHUNK, M_HALF, FC), jnp.bfloat16),
            pltpu.SemaphoreType.DMA((N_CHUNK,)),
            pltpu.SemaphoreType.DMA((N_CHUNK,)),
        ],
        compiler_params=pltpu.CompilerParams(
            collective_id=0, vmem_limit_bytes=100 * 1024 * 1024
        ),
    )(x, dy)
